# baseline (speedup 1.0000x reference)
import functools

import jax
import jax.numpy as jnp
from jax import lax
from jax.experimental import pallas as pl
from jax.experimental.pallas import tpu as pltpu
from jax.experimental.pallas import tpu_sc as plsc

_B = 16384
_D = 64
_NC = 2
_NS = 16
_NW = _NC * _NS
_CHUNK = _B // _NW
_G = 16
_H = _CHUNK // 2
_NGROUPS = _H // _G


def _als_body(uid_hbm, iid_hbm, u2_hbm, i2_hbm, ub_hbm, ib_hbm, out_hbm,
              uid_v, iid_v, uix_v, iix_v, ublk, iblk, ub_v, ib_v, out_v,
              sem_ids, sem_b, sem_u, sem_i):
    wid = lax.axis_index("s") * _NC + lax.axis_index("c")
    base = wid * _CHUNK

    cp_uid = pltpu.async_copy(uid_hbm.at[pl.ds(base, _CHUNK)], uid_v, sem_ids)
    cp_iid = pltpu.async_copy(iid_hbm.at[pl.ds(base, _CHUNK)], iid_v, sem_ids)
    cp_uid.wait()
    cp_iid.wait()
    for k in range(_CHUNK // 16):
        uix_v[pl.ds(k * 16, 16)] = uid_v[pl.ds(k * 16, 16)] >> 1
        iix_v[pl.ds(k * 16, 16)] = iid_v[pl.ds(k * 16, 16)] >> 1

    cp_ub = pltpu.async_copy(ub_hbm.at[uid_v], ub_v, sem_b)
    cp_ib = pltpu.async_copy(ib_hbm.at[iid_v], ib_v, sem_b)
    cp_ub.wait()
    cp_ib.wait()

    lanes = lax.iota(jnp.int32, 16)

    for h in range(2):
        cp_u = pltpu.async_copy(u2_hbm.at[uix_v.at[pl.ds(h * _H, _H)]],
                                ublk, sem_u)
        cp_i = pltpu.async_copy(i2_hbm.at[iix_v.at[pl.ds(h * _H, _H)]],
                                iblk, sem_i)
        cp_u.wait()
        cp_i.wait()

        def group(g, carry, h=h):
            uo16 = (uid_v[pl.ds(h * _H + g * 16, 16)] & 1) * _D
            io16 = (iid_v[pl.ds(h * _H + g * 16, 16)] & 1) * _D
            tot = jnp.zeros((16,), jnp.float32)
            for j in range(_G):
                row = g * _G + j
                uo = uo16[j]
                io = io16[j]
                acc = jnp.zeros((16,), jnp.float32)
                for c in range(_D // 16):
                    acc = acc + (ublk[row, pl.ds(uo + c * 16, 16)]
                                 * iblk[row, pl.ds(io + c * 16, 16)])
                tot = jnp.where(lanes == j, jnp.sum(acc), tot)
            off = h * _H + g * 16
            tot = tot + ub_v[pl.ds(off, 16)] + ib_v[pl.ds(off, 16)]
            out_v[pl.ds(off, 16)] = tot
            return carry

        lax.fori_loop(0, _NGROUPS, group, 0)

    pltpu.sync_copy(out_v, out_hbm.at[pl.ds(base, _CHUNK)])


_als = functools.partial(
    pl.kernel,
    out_type=jax.ShapeDtypeStruct((_B,), jnp.float32),
    mesh=plsc.VectorSubcoreMesh(core_axis_name="c", subcore_axis_name="s",
                                num_cores=_NC, num_subcores=_NS),
    compiler_params=pltpu.CompilerParams(needs_layout_passes=False,
                                         use_tc_tiling_on_sc=True),
    scratch_types=[
        pltpu.VMEM((_CHUNK,), jnp.int32),
        pltpu.VMEM((_CHUNK,), jnp.int32),
        pltpu.VMEM((_CHUNK,), jnp.int32),
        pltpu.VMEM((_CHUNK,), jnp.int32),
        pltpu.VMEM((_H, 2 * _D), jnp.float32),
        pltpu.VMEM((_H, 2 * _D), jnp.float32),
        pltpu.VMEM((_CHUNK,), jnp.float32),
        pltpu.VMEM((_CHUNK,), jnp.float32),
        pltpu.VMEM((_CHUNK,), jnp.float32),
        pltpu.SemaphoreType.DMA,
        pltpu.SemaphoreType.DMA,
        pltpu.SemaphoreType.DMA,
        pltpu.SemaphoreType.DMA,
    ],
)(_als_body)


def kernel(user_id, item_id, u, i, u_bias, i_bias):
    n2 = u.shape[0] // 2
    return _als(user_id.astype(jnp.int32), item_id.astype(jnp.int32),
                u.reshape(n2, 2 * u.shape[1]), i.reshape(n2, 2 * i.shape[1]),
                u_bias, i_bias)
